# Initial kernel scaffold; baseline (speedup 1.0000x reference)
import functools

import jax
import jax.numpy as jnp
from jax import lax
from jax.experimental import pallas as pl
from jax.experimental.pallas import tpu as pltpu
from jax.experimental.pallas import tpu_sc as plsc

_N = 10000
_E = 320000
_K = 125
_CHUNKS = _E // _K
_NCORES = 2
_NSUB = 16
_NW = _NCORES * _NSUB
_CPW = _CHUNKS // _NW
_RPS = _N // _NSUB
_ZR = 125
_DEGW = 16

_mesh = plsc.VectorSubcoreMesh(core_axis_name="c", subcore_axis_name="s")


def _wid(cid, sid):
    return sid * _NCORES + cid


def _deg_body(dst2d, ones_hbm, zdeg_hbm, degp, dst_idx, ones_v, zdeg_v, accum):
    cid = lax.axis_index("c")
    sid = lax.axis_index("s")
    w = _wid(cid, sid)
    pltpu.sync_copy(dst2d.at[pl.ds(w * _CPW, _CPW)], dst_idx)
    pltpu.sync_copy(ones_hbm, ones_v)
    pltpu.sync_copy(zdeg_hbm, zdeg_v)
    pltpu.sync_copy(zdeg_v, accum.at[pl.ds(sid * _RPS, _RPS)])
    plsc.subcore_barrier()

    def body(j, _):
        pltpu.sync_copy(ones_v, accum.at[dst_idx.at[j]], add=True)
        return ()

    lax.fori_loop(0, _CPW, body, ())
    plsc.subcore_barrier()
    pltpu.sync_copy(accum.at[pl.ds(sid * _RPS, _RPS)],
                    degp.at[pl.ds(cid * _N + sid * _RPS, _RPS)])


_deg_call = pl.kernel(
    _deg_body,
    out_type=jax.ShapeDtypeStruct((2 * _N, _DEGW), jnp.float32),
    mesh=_mesh,
    scratch_types=[
        pltpu.VMEM((_CPW, _K), jnp.int32),
        pltpu.VMEM((_K, _DEGW), jnp.float32),
        pltpu.VMEM((_RPS, _DEGW), jnp.float32),
        pltpu.VMEM_SHARED((_N, _DEGW), jnp.float32),
    ],
)


def _agg_body(y_hbm, src2d, dst2d, zeros_hbm, out_hbm,
              src_idx, dst_idx, rows, zeros_v, accum, sem0, sem1):
    cid = lax.axis_index("c")
    sid = lax.axis_index("s")
    w = _wid(cid, sid)
    pltpu.sync_copy(src2d.at[pl.ds(w * _CPW, _CPW)], src_idx)
    pltpu.sync_copy(dst2d.at[pl.ds(w * _CPW, _CPW)], dst_idx)
    pltpu.sync_copy(zeros_hbm, zeros_v)
    for c in range(_RPS // _ZR):
        pltpu.sync_copy(zeros_v, accum.at[pl.ds(sid * _RPS + c * _ZR, _ZR)])
    plsc.subcore_barrier()

    def body(jj, _):
        j0 = jj * 2
        j1 = j0 + 1
        pltpu.async_copy(y_hbm.at[src_idx.at[j0]], rows.at[0], sem0)
        pltpu.async_copy(y_hbm.at[src_idx.at[j1]], rows.at[1], sem1)
        pltpu.make_async_copy(y_hbm.at[src_idx.at[j0]], rows.at[0], sem0).wait()
        pltpu.sync_copy(rows.at[0], accum.at[dst_idx.at[j0]], add=True)
        pltpu.make_async_copy(y_hbm.at[src_idx.at[j1]], rows.at[1], sem1).wait()
        pltpu.sync_copy(rows.at[1], accum.at[dst_idx.at[j1]], add=True)
        return ()

    lax.fori_loop(0, _CPW // 2, body, ())
    plsc.subcore_barrier()
    for c in range(_RPS // _ZR):
        pltpu.sync_copy(
            accum.at[pl.ds(sid * _RPS + c * _ZR, _ZR)],
            out_hbm.at[pl.ds(cid * _N + sid * _RPS + c * _ZR, _ZR)])


def _make_agg(d):
    return pl.kernel(
        _agg_body,
        out_type=jax.ShapeDtypeStruct((2 * _N, d), jnp.float32),
        mesh=_mesh,
        scratch_types=[
            pltpu.VMEM((_CPW, _K), jnp.int32),
            pltpu.VMEM((_CPW, _K), jnp.int32),
            pltpu.VMEM((2, _K, d), jnp.float32),
            pltpu.VMEM((_ZR, d), jnp.float32),
            pltpu.VMEM_SHARED((_N, d), jnp.float32),
            pltpu.SemaphoreType.DMA,
            pltpu.SemaphoreType.DMA,
        ],
    )


_agg128 = _make_agg(128)
_agg48 = _make_agg(48)


def _dinv_of(degp_ref):
    deg = degp_ref[0:_N, 0:1] + degp_ref[_N:2 * _N, 0:1] + 1.0
    return lax.rsqrt(jnp.maximum(deg, 1.0))


def _stage1_body(x_ref, w_ref, degp_ref, y_ref):
    h = jnp.dot(x_ref[...], w_ref[...], preferred_element_type=jnp.float32)
    y_ref[...] = h * _dinv_of(degp_ref)


def _stage_mid_body(s_ref, y_ref, degp_ref, b_ref, gam_ref, bet_ref, w_ref,
                    o_ref):
    dinv = _dinv_of(degp_ref)
    agg = (s_ref[0:_N, :] + s_ref[_N:2 * _N, :] + y_ref[...]) * dinv + b_ref[...]
    mean = jnp.mean(agg, axis=0, keepdims=True)
    cen = agg - mean
    var = jnp.mean(cen * cen, axis=0, keepdims=True)
    xn = cen * lax.rsqrt(var + 1e-5) * gam_ref[...] + bet_ref[...]
    h = jnp.maximum(xn, 0.0)
    o_ref[...] = jnp.dot(h, w_ref[...],
                         preferred_element_type=jnp.float32) * dinv


def _stage4_body(s_ref, y_ref, degp_ref, b_ref, o_ref):
    dinv = _dinv_of(degp_ref)
    agg = (s_ref[0:_N, :] + s_ref[_N:2 * _N, :] + y_ref[...]) * dinv
    logits = agg[:, 0:40] + b_ref[...]
    m = jnp.max(logits, axis=-1, keepdims=True)
    sh = logits - m
    lse = jnp.log(jnp.sum(jnp.exp(sh), axis=-1, keepdims=True))
    o_ref[...] = sh - lse


_stage1 = pl.pallas_call(
    _stage1_body, out_shape=jax.ShapeDtypeStruct((_N, 128), jnp.float32))


def _mk_mid(d_out):
    return pl.pallas_call(
        _stage_mid_body,
        out_shape=jax.ShapeDtypeStruct((_N, d_out), jnp.float32))


_stage2 = _mk_mid(128)
_stage3 = _mk_mid(48)

_stage4 = pl.pallas_call(
    _stage4_body, out_shape=jax.ShapeDtypeStruct((_N, 40), jnp.float32))


def kernel(x, adj_t, W0, b0, g0, be0, W1, b1, g1, be1, W2, b2):
    src2d = adj_t[0].reshape(_CHUNKS, _K)
    dst2d = adj_t[1].reshape(_CHUNKS, _K)
    ones_deg = jnp.ones((_K, _DEGW), jnp.float32)
    zeros_deg = jnp.zeros((_RPS, _DEGW), jnp.float32)
    zeros128 = jnp.zeros((_ZR, 128), jnp.float32)
    zeros48 = jnp.zeros((_ZR, 48), jnp.float32)
    w2p = jnp.pad(W2, ((0, 0), (0, 8)))

    degp = _deg_call(dst2d, ones_deg, zeros_deg)
    y0 = _stage1(x, W0, degp)
    s0 = _agg128(y0, src2d, dst2d, zeros128)
    y1 = _stage2(s0, y0, degp, b0.reshape(1, 128), g0.reshape(1, 128),
                 be0.reshape(1, 128), W1)
    s1 = _agg128(y1, src2d, dst2d, zeros128)
    y2 = _stage3(s1, y1, degp, b1.reshape(1, 128), g1.reshape(1, 128),
                 be1.reshape(1, 128), w2p)
    s2 = _agg48(y2, src2d, dst2d, zeros48)
    return _stage4(s2, y2, degp, b2.reshape(1, 40))


# speedup vs baseline: 17.9949x; 17.9949x over previous
import functools

import jax
import jax.numpy as jnp
from jax import lax
from jax.experimental import pallas as pl
from jax.experimental.pallas import tpu as pltpu
from jax.experimental.pallas import tpu_sc as plsc

_N = 10000
_E = 320000
_K = 125
_CHUNKS = _E // _K
_NCORES = 2
_NSUB = 16
_NW = _NCORES * _NSUB
_CPW = _CHUNKS // _NW
_NPAD = 10240
_RPS = _NPAD // _NSUB
_XR = 64
_GRP = 16
_DEGW = 16

@functools.cache
def _get_mesh():
    return plsc.VectorSubcoreMesh(core_axis_name="c", subcore_axis_name="s",
                                  num_cores=_NCORES, num_subcores=_NSUB)


def _wid(cid, sid):
    return sid * _NCORES + cid


def _deg_body(dst2d, ones_hbm, zeros_hbm, riota_hbm, degp,
              dst_idx, ones_v, zeros_v, xfer,
              ridx0, ridx1, ridx2, ridx3, ridx4, accum):
    cid = lax.axis_index("c")
    sid = lax.axis_index("s")
    w = _wid(cid, sid)
    ridxs = (ridx0, ridx1, ridx2, ridx3, ridx4)
    pltpu.sync_copy(dst2d.at[pl.ds(w * _CPW, _CPW)], dst_idx)
    pltpu.sync_copy(ones_hbm, ones_v)
    pltpu.sync_copy(zeros_hbm, zeros_v)
    for c in range(_RPS // 128):
        pltpu.sync_copy(
            riota_hbm.at[pl.ds(sid * _RPS + c * 128, 128)], ridxs[c])
    for c in range(_RPS // 128):
        pltpu.sync_copy(zeros_v, accum.at[ridxs[c]])
    plsc.subcore_barrier()

    for j in range(_CPW):
        pltpu.sync_copy(ones_v, accum.at[dst_idx.at[j]], add=True)
    plsc.subcore_barrier()
    for c in range(_RPS // 128):
        pltpu.sync_copy(accum.at[ridxs[c]], xfer)
        pltpu.sync_copy(
            xfer, degp.at[pl.ds(cid * _NPAD + sid * _RPS + c * 128, 128)])


@functools.cache
def _get_deg_call():
    return pl.kernel(
        _deg_body,
        out_type=jax.ShapeDtypeStruct((2 * _NPAD, _DEGW), jnp.float32),
        mesh=_get_mesh(),
        scratch_types=[
            pltpu.VMEM((_CPW, _K), jnp.int32),
            pltpu.VMEM((_K, _DEGW), jnp.float32),
            pltpu.VMEM((128, _DEGW), jnp.float32),
            pltpu.VMEM((128, _DEGW), jnp.float32),
            pltpu.VMEM((128,), jnp.int32),
            pltpu.VMEM((128,), jnp.int32),
            pltpu.VMEM((128,), jnp.int32),
            pltpu.VMEM((128,), jnp.int32),
            pltpu.VMEM((128,), jnp.int32),
            pltpu.VMEM_SHARED((_NPAD, _DEGW), jnp.float32),
        ],
    )


def _agg_body(y_hbm, src2d, dst2d, zeros_hbm, riota_hbm, out_hbm,
              src_idx, dst_idx, rows, zx,
              ri0, ri1, ri2, ri3, ri4, ri5, ri6, ri7, ri8, ri9,
              accum, sem0):
    cid = lax.axis_index("c")
    sid = lax.axis_index("s")
    w = _wid(cid, sid)
    ridxs = (ri0, ri1, ri2, ri3, ri4, ri5, ri6, ri7, ri8, ri9)
    pltpu.sync_copy(zeros_hbm, zx)
    for c in range(_RPS // _XR):
        pltpu.sync_copy(
            riota_hbm.at[pl.ds(sid * _RPS + c * _XR, _XR)], ridxs[c])
        pltpu.sync_copy(zx, accum.at[ridxs[c]])
    plsc.subcore_barrier()

    for g in range(_CPW // _GRP):
        start = w * _CPW + g * _GRP
        pltpu.sync_copy(src2d.at[pl.ds(start, _GRP)], src_idx)
        pltpu.sync_copy(dst2d.at[pl.ds(start, _GRP)], dst_idx)
        for j in range(_GRP):
            pltpu.sync_copy(y_hbm.at[src_idx.at[j]], rows.at[0])
            pltpu.sync_copy(rows.at[0], accum.at[dst_idx.at[j]], add=True)
    plsc.subcore_barrier()
    for c in range(_RPS // _XR):
        pltpu.sync_copy(accum.at[ridxs[c]], zx)
        pltpu.sync_copy(
            zx, out_hbm.at[pl.ds(cid * _NPAD + sid * _RPS + c * _XR, _XR)])


@functools.cache
def _make_agg(d):
    return pl.kernel(
        _agg_body,
        out_type=jax.ShapeDtypeStruct((2 * _NPAD, d), jnp.float32),
        mesh=_get_mesh(),
        scratch_types=[
            pltpu.VMEM((_GRP, _K), jnp.int32),
            pltpu.VMEM((_GRP, _K), jnp.int32),
            pltpu.VMEM((2, _K, d), jnp.float32),
            pltpu.VMEM((_XR, d), jnp.float32),
            pltpu.VMEM((_XR,), jnp.int32),
            pltpu.VMEM((_XR,), jnp.int32),
            pltpu.VMEM((_XR,), jnp.int32),
            pltpu.VMEM((_XR,), jnp.int32),
            pltpu.VMEM((_XR,), jnp.int32),
            pltpu.VMEM((_XR,), jnp.int32),
            pltpu.VMEM((_XR,), jnp.int32),
            pltpu.VMEM((_XR,), jnp.int32),
            pltpu.VMEM((_XR,), jnp.int32),
            pltpu.VMEM((_XR,), jnp.int32),
            pltpu.VMEM_SHARED((_NPAD, d), jnp.float32),
            pltpu.SemaphoreType.DMA,
        ],
    )


def _dinv_of(degp_ref):
    deg = degp_ref[0:_N, 0:1] + degp_ref[_NPAD:_NPAD + _N, 0:1] + 1.0
    return lax.rsqrt(jnp.maximum(deg, 1.0))


def _stage1_body(x_ref, w_ref, degp_ref, y_ref):
    h = jnp.dot(x_ref[...], w_ref[...], preferred_element_type=jnp.float32)
    y_ref[...] = h * _dinv_of(degp_ref)


def _stage_mid_body(s_ref, y_ref, degp_ref, b_ref, gam_ref, bet_ref, w_ref,
                    o_ref):
    dinv = _dinv_of(degp_ref)
    agg = (s_ref[0:_N, :] + s_ref[_NPAD:_NPAD + _N, :] + y_ref[...]) * dinv + b_ref[...]
    mean = jnp.mean(agg, axis=0, keepdims=True)
    cen = agg - mean
    var = jnp.mean(cen * cen, axis=0, keepdims=True)
    xn = cen * lax.rsqrt(var + 1e-5) * gam_ref[...] + bet_ref[...]
    h = jnp.maximum(xn, 0.0)
    o_ref[...] = jnp.dot(h, w_ref[...],
                         preferred_element_type=jnp.float32) * dinv


def _stage4_body(s_ref, y_ref, degp_ref, b_ref, o_ref):
    dinv = _dinv_of(degp_ref)
    agg = (s_ref[0:_N, :] + s_ref[_NPAD:_NPAD + _N, :] + y_ref[...]) * dinv
    logits = agg[:, 0:40] + b_ref[...]
    m = jnp.max(logits, axis=-1, keepdims=True)
    sh = logits - m
    lse = jnp.log(jnp.sum(jnp.exp(sh), axis=-1, keepdims=True))
    o_ref[...] = sh - lse


_stage1 = pl.pallas_call(
    _stage1_body, out_shape=jax.ShapeDtypeStruct((_N, 128), jnp.float32))


def _mk_mid(d_out):
    return pl.pallas_call(
        _stage_mid_body,
        out_shape=jax.ShapeDtypeStruct((_N, d_out), jnp.float32))


_stage2 = _mk_mid(128)
_stage3 = _mk_mid(128)

_stage4 = pl.pallas_call(
    _stage4_body, out_shape=jax.ShapeDtypeStruct((_N, 40), jnp.float32))


def _ref_math(x, adj_t, W0, b0, g0, be0, W1, b1, g1, be1, W2, b2):
    n = x.shape[0]
    src = adj_t[0]
    dst = adj_t[1]
    deg = jax.ops.segment_sum(jnp.ones(src.shape, x.dtype), dst,
                              num_segments=n) + 1.0
    dinv = lax.rsqrt(deg)

    def conv(h, W, b):
        y = (h @ W) * dinv[:, None]
        S = jax.ops.segment_sum(jnp.take(y, src, axis=0), dst, num_segments=n)
        return (S + y) * dinv[:, None] + b

    def bn(h, gam, bet):
        mean = jnp.mean(h, axis=0)
        var = jnp.var(h, axis=0)
        return (h - mean) * lax.rsqrt(var + 1e-5) * gam + bet

    h = jax.nn.relu(bn(conv(x, W0, b0), g0, be0))
    h = jax.nn.relu(bn(conv(h, W1, b1), g1, be1))
    h = conv(h, W2, b2)
    return jax.nn.log_softmax(h, axis=-1)


def kernel(x, adj_t, W0, b0, g0, be0, W1, b1, g1, be1, W2, b2):
    src2d = adj_t[0].reshape(_CHUNKS, _K)
    dst2d = adj_t[1].reshape(_CHUNKS, _K)
    ones_deg = jnp.ones((_K, _DEGW), jnp.float32)
    zeros_deg = jnp.zeros((128, _DEGW), jnp.float32)
    riota_deg = jnp.arange(_NPAD, dtype=jnp.int32)
    riota_agg = jnp.arange(_NPAD, dtype=jnp.int32)
    zeros128 = jnp.zeros((_XR, 128), jnp.float32)
    w2p = jnp.pad(W2, ((0, 0), (0, 88)))

    agg128 = _make_agg(128)

    degp = _get_deg_call()(dst2d, ones_deg, zeros_deg, riota_deg)
    y0 = _stage1(x, W0, degp)
    s0 = agg128(y0, src2d, dst2d, zeros128, riota_agg)
    y1 = _stage2(s0, y0, degp, b0.reshape(1, 128), g0.reshape(1, 128),
                 be0.reshape(1, 128), W1)
    s1 = agg128(y1, src2d, dst2d, zeros128, riota_agg)
    y2 = _stage3(s1, y1, degp, b1.reshape(1, 128), g1.reshape(1, 128),
                 be1.reshape(1, 128), w2p)
    s2 = agg128(y2, src2d, dst2d, zeros128, riota_agg)
    return _stage4(s2, y2, degp, b2.reshape(1, 40))
